# Initial kernel scaffold; baseline (speedup 1.0000x reference)
import jax
import jax.numpy as jnp
from jax.experimental import pallas as pl


def kernel(x, Z, scale_param, shift_param):
    raise NotImplementedError("write your pallas kernel here")


# speedup vs baseline: 177.0183x; 177.0183x over previous
import functools

import jax
import jax.numpy as jnp
from jax import lax
from jax.experimental import pallas as pl
from jax.experimental.pallas import tpu as pltpu
from jax.experimental.pallas import tpu_sc as plsc

N = 4_000_000
CHUNK = 16_000
NUM_CHUNKS = N // CHUNK
L = 16
NC, NS = 2, 16
NW = NC * NS
TBL = 128


@functools.partial(
    pl.kernel,
    out_type=jax.ShapeDtypeStruct((N,), jnp.float32),
    mesh=plsc.VectorSubcoreMesh(core_axis_name="c", subcore_axis_name="s"),
    scratch_types=[
        pltpu.VMEM((CHUNK,), jnp.float32),
        pltpu.VMEM((CHUNK,), jnp.int32),
        pltpu.VMEM((CHUNK,), jnp.float32),
        pltpu.VMEM((TBL,), jnp.float32),
        pltpu.VMEM((TBL,), jnp.float32),
    ],
    compiler_params=pltpu.CompilerParams(needs_layout_passes=False),
)
def _scale_shift_sc(x_hbm, z_hbm, s_hbm, b_hbm, out_hbm, xv, zv, ov, ts, tb):
    wid = lax.axis_index("s") * NC + lax.axis_index("c")
    pltpu.sync_copy(s_hbm, ts)
    pltpu.sync_copy(b_hbm, tb)

    @pl.loop(wid, NUM_CHUNKS, step=NW)
    def _chunk(c):
        off = c * CHUNK
        pltpu.sync_copy(x_hbm.at[pl.ds(off, CHUNK)], xv)
        pltpu.sync_copy(z_hbm.at[pl.ds(off, CHUNK)], zv)

        @pl.loop(0, CHUNK // L)
        def _vec(i):
            sl = pl.ds(i * L, L)
            idx = zv[sl]
            s = plsc.load_gather(ts, [idx])
            b = plsc.load_gather(tb, [idx])
            ov[sl] = s * xv[sl] + b

        pltpu.sync_copy(ov, out_hbm.at[pl.ds(off, CHUNK)])


def kernel(x, Z, scale_param, shift_param):
    xf = x.reshape(N).astype(jnp.float32)
    zi = Z.astype(jnp.int32)
    ts = jnp.pad(scale_param.reshape(-1).astype(jnp.float32),
                 (0, TBL - scale_param.shape[0]))
    tb = jnp.pad(shift_param.reshape(-1).astype(jnp.float32),
                 (0, TBL - shift_param.shape[0]))
    out = _scale_shift_sc(xf, zi, ts, tb)
    return out.reshape(N, 1)


# speedup vs baseline: 199.7016x; 1.1281x over previous
import functools

import jax
import jax.numpy as jnp
from jax import lax
from jax.experimental import pallas as pl
from jax.experimental.pallas import tpu as pltpu
from jax.experimental.pallas import tpu_sc as plsc

N = 4_000_000
CHUNK = 16_000
NUM_CHUNKS = N // CHUNK
L = 16
NC, NS = 2, 16
NW = NC * NS
TBL = 128


@functools.partial(
    pl.kernel,
    out_type=jax.ShapeDtypeStruct((N,), jnp.float32),
    mesh=plsc.VectorSubcoreMesh(core_axis_name="c", subcore_axis_name="s"),
    scratch_types=[
        pltpu.VMEM((CHUNK,), jnp.float32),
        pltpu.VMEM((CHUNK,), jnp.int32),
        pltpu.VMEM((CHUNK,), jnp.float32),
        pltpu.VMEM((TBL,), jnp.float32),
        pltpu.VMEM((TBL,), jnp.float32),
    ],
    compiler_params=pltpu.CompilerParams(needs_layout_passes=False),
)
def _scale_shift_sc(x_hbm, z_hbm, s_hbm, b_hbm, out_hbm, xv, zv, ov, ts, tb):
    wid = lax.axis_index("s") * NC + lax.axis_index("c")
    pltpu.sync_copy(s_hbm, ts)
    pltpu.sync_copy(b_hbm, tb)

    @pl.loop(wid, NUM_CHUNKS, step=NW)
    def _chunk(c):
        off = c * CHUNK
        pltpu.sync_copy(x_hbm.at[pl.ds(off, CHUNK)], xv)
        pltpu.sync_copy(z_hbm.at[pl.ds(off, CHUNK)], zv)

        @plsc.parallel_loop(0, CHUNK // L, unroll=8)
        def _vec(i):
            sl = pl.ds(i * L, L)
            idx = zv[sl]
            s = plsc.load_gather(ts, [idx])
            b = plsc.load_gather(tb, [idx])
            ov[sl] = s * xv[sl] + b

        pltpu.sync_copy(ov, out_hbm.at[pl.ds(off, CHUNK)])


def kernel(x, Z, scale_param, shift_param):
    xf = x.reshape(N).astype(jnp.float32)
    zi = Z.astype(jnp.int32)
    ts = jnp.pad(scale_param.reshape(-1).astype(jnp.float32),
                 (0, TBL - scale_param.shape[0]))
    tb = jnp.pad(shift_param.reshape(-1).astype(jnp.float32),
                 (0, TBL - shift_param.shape[0]))
    out = _scale_shift_sc(xf, zi, ts, tb)
    return out.reshape(N, 1)


# speedup vs baseline: 663.7749x; 3.3238x over previous
import functools

import jax
import jax.numpy as jnp
from jax import lax
from jax.experimental import pallas as pl
from jax.experimental.pallas import tpu as pltpu
from jax.experimental.pallas import tpu_sc as plsc

N = 4_000_000
CHUNK = 6_400
NUM_CHUNKS = N // CHUNK
L = 16
NC, NS = 2, 16
NW = NC * NS
N_SP = 119


@functools.partial(
    pl.kernel,
    out_type=jax.ShapeDtypeStruct((1, N), jnp.float32),
    mesh=plsc.VectorSubcoreMesh(core_axis_name="c", subcore_axis_name="s"),
    scratch_types=[
        pltpu.VMEM((1, CHUNK), jnp.float32),
        pltpu.VMEM((CHUNK,), jnp.int32),
        pltpu.VMEM((1, CHUNK), jnp.float32),
        pltpu.VMEM((1, N_SP), jnp.float32),
        pltpu.VMEM((1, N_SP), jnp.float32),
    ],
    compiler_params=pltpu.CompilerParams(
        needs_layout_passes=False, use_tc_tiling_on_sc=True),
)
def _scale_shift_sc(x_hbm, z_hbm, s_hbm, b_hbm, out_hbm, xv, zv, ov, ts, tb):
    wid = lax.axis_index("s") * NC + lax.axis_index("c")
    pltpu.sync_copy(s_hbm, ts)
    pltpu.sync_copy(b_hbm, tb)
    zero = jnp.zeros((L,), jnp.int32)

    @pl.loop(wid, NUM_CHUNKS, step=NW)
    def _chunk(c):
        off = c * CHUNK
        pltpu.sync_copy(x_hbm.at[:, pl.ds(off, CHUNK)], xv)
        pltpu.sync_copy(z_hbm.at[pl.ds(off, CHUNK)], zv)

        @plsc.parallel_loop(0, CHUNK // L, unroll=8)
        def _vec(i):
            sl = pl.ds(i * L, L)
            idx = zv[sl]
            s = plsc.load_gather(ts, [zero, idx])
            b = plsc.load_gather(tb, [zero, idx])
            ov[0, sl] = s * xv[0, sl] + b

        pltpu.sync_copy(ov, out_hbm.at[:, pl.ds(off, CHUNK)])


def kernel(x, Z, scale_param, shift_param):
    out = _scale_shift_sc(x.T, Z.astype(jnp.int32), scale_param.T,
                          shift_param.T)
    return out.T


# speedup vs baseline: 1076.3575x; 1.6216x over previous
import functools

import jax
import jax.numpy as jnp
from jax import lax
from jax.experimental import pallas as pl
from jax.experimental.pallas import tpu as pltpu
from jax.experimental.pallas import tpu_sc as plsc

N = 4_000_000
CHUNK = 3_200
NUM_CHUNKS = N // CHUNK
L = 16
NC, NS = 2, 16
NW = NC * NS
N_SP = 119
NK_MAX = -(-NUM_CHUNKS // NW)
NK_EVEN = NK_MAX + (NK_MAX % 2)


@functools.partial(
    pl.kernel,
    out_type=jax.ShapeDtypeStruct((1, N), jnp.float32),
    mesh=plsc.VectorSubcoreMesh(core_axis_name="c", subcore_axis_name="s"),
    scratch_types=[
        pltpu.VMEM((1, CHUNK), jnp.float32),
        pltpu.VMEM((1, CHUNK), jnp.float32),
        pltpu.VMEM((CHUNK,), jnp.int32),
        pltpu.VMEM((CHUNK,), jnp.int32),
        pltpu.VMEM((1, CHUNK), jnp.float32),
        pltpu.VMEM((1, CHUNK), jnp.float32),
        pltpu.VMEM((1, N_SP), jnp.float32),
        pltpu.VMEM((1, N_SP), jnp.float32),
        pltpu.SemaphoreType.DMA,
        pltpu.SemaphoreType.DMA,
        pltpu.SemaphoreType.DMA,
        pltpu.SemaphoreType.DMA,
        pltpu.SemaphoreType.DMA,
        pltpu.SemaphoreType.DMA,
    ],
    compiler_params=pltpu.CompilerParams(
        needs_layout_passes=False, use_tc_tiling_on_sc=True),
)
def _scale_shift_sc(x_hbm, z_hbm, s_hbm, b_hbm, out_hbm,
                    xv0, xv1, zv0, zv1, ov0, ov1, ts, tb,
                    sx0, sx1, sz0, sz1, so0, so1):
    xv, zv, ov = (xv0, xv1), (zv0, zv1), (ov0, ov1)
    sx, sz, so = (sx0, sx1), (sz0, sz1), (so0, so1)
    wid = lax.axis_index("s") * NC + lax.axis_index("c")
    nk = (NUM_CHUNKS - wid + NW - 1) // NW
    pltpu.sync_copy(s_hbm, ts)
    pltpu.sync_copy(b_hbm, tb)
    zero = jnp.zeros((L,), jnp.int32)

    def start_in(k, b):
        off = (wid + k * NW) * CHUNK
        pltpu.async_copy(x_hbm.at[:, pl.ds(off, CHUNK)], xv[b], sx[b])
        pltpu.async_copy(z_hbm.at[pl.ds(off, CHUNK)], zv[b], sz[b])

    def wait_in(b):
        pltpu.make_async_copy(x_hbm.at[:, pl.ds(0, CHUNK)], xv[b], sx[b]).wait()
        pltpu.make_async_copy(z_hbm.at[pl.ds(0, CHUNK)], zv[b], sz[b]).wait()

    def start_out(k, b):
        off = (wid + k * NW) * CHUNK
        pltpu.async_copy(ov[b], out_hbm.at[:, pl.ds(off, CHUNK)], so[b])

    def wait_out(b):
        pltpu.make_async_copy(ov[b], out_hbm.at[:, pl.ds(0, CHUNK)], so[b]).wait()

    start_in(0, 0)

    @pl.when(nk > 1)
    def _():
        start_in(1, 1)

    @pl.loop(0, NK_EVEN, step=2)
    def _pair(kk):
        for b in (0, 1):
            k = kk + b

            @pl.when(k < nk)
            def _():
                wait_in(b)

                @pl.when(k >= 2)
                def _():
                    wait_out(b)

                @plsc.parallel_loop(0, CHUNK // L, unroll=8)
                def _vec(i):
                    sl = pl.ds(i * L, L)
                    idx = zv[b][sl]
                    s = plsc.load_gather(ts, [zero, idx])
                    sh = plsc.load_gather(tb, [zero, idx])
                    ov[b][0, sl] = s * xv[b][0, sl] + sh

                start_out(k, b)

                @pl.when(k + 2 < nk)
                def _():
                    start_in(k + 2, b)

    for b in (0, 1):
        @pl.when(nk > b)
        def _():
            wait_out(b)


def kernel(x, Z, scale_param, shift_param):
    out = _scale_shift_sc(x.T, Z.astype(jnp.int32), scale_param.T,
                          shift_param.T)
    return out.T
